# baseline (speedup 1.0000x reference)
import functools

import jax
import jax.numpy as jnp
from jax import lax
from jax.experimental import pallas as pl
from jax.experimental.pallas import tpu as pltpu

_NUM_CLASSES = 21
_THRESH = 0.45
_P = 8732
_T = 8
_B = 16
_LANES = 128
_P_PAD = ((_P + _LANES - 1) // _LANES) * _LANES
_BIG = 2**30
_BPS = 8

_DN = (((0,), (0,)), ((), ()))


def _one_batch(bb, u, tb_ref, tc_ref, pri_ref, loc_ref, conf_ref,
               lg_ref, np_ref):
    tbb = tb_ref[u]
    tx1 = tbb[:, 0:1]
    ty1 = tbb[:, 1:2]
    tx2 = tbb[:, 2:3]
    ty2 = tbb[:, 3:4]
    labf = tc_ref[u][:, 0:1]

    tw = tx2 - tx1
    th = ty2 - ty1
    area_t = tw * th
    qt = jnp.concatenate(
        [(tx1 + tx2) * 0.5, (ty1 + ty2) * 0.5,
         5.0 * jnp.log(tw), 5.0 * jnp.log(th), labf], axis=1)

    pp = pri_ref[...]
    px1 = pp[0:1, :]
    py1 = pp[1:2, :]
    px2 = pp[2:3, :]
    py2 = pp[3:4, :]
    area_p = pp[4:5, :]
    pcx = pp[5:6, :]
    pcy = pp[6:7, :]
    inv01w = pp[7:8, :]
    inv01h = pp[8:9, :]
    lpw5 = pp[9:10, :]
    lph5 = pp[10:11, :]

    iw = jnp.maximum(jnp.minimum(tx2, px2) - jnp.maximum(tx1, px1), 0.0)
    ih = jnp.maximum(jnp.minimum(ty2, py2) - jnp.maximum(ty1, py1), 0.0)
    inter = iw * ih
    ov = inter / (area_t + area_p - inter)

    iota_p = lax.broadcasted_iota(jnp.int32, (1, _P_PAD), 1)
    iota_t = lax.broadcasted_iota(jnp.int32, (_T, 1), 0)

    bp_max = jnp.max(ov, axis=1, keepdims=True)
    bp_idx = jnp.min(jnp.where(ov == bp_max, iota_p, _BIG),
                     axis=1, keepdims=True)
    bt_ov = jnp.max(ov, axis=0, keepdims=True)
    bt_idx = jnp.min(jnp.where(ov == bt_ov, iota_t, _BIG),
                     axis=0, keepdims=True)

    eq = bp_idx == iota_p
    forced = jnp.max(jnp.where(eq, iota_t, -1), axis=0, keepdims=True)
    bt_ov = jnp.where(forced >= 0, 2.0, bt_ov)
    bt_idx = jnp.where(forced >= 0, forced, bt_idx)

    onehot = jnp.where(bt_idx == iota_t, 1.0, 0.0)
    m = lax.dot_general(qt, onehot, _DN,
                        preferred_element_type=jnp.float32)
    m_lab = jnp.floor(m[4:5, :] + 0.5)
    conf_f = jnp.where(bt_ov < _THRESH, 0.0, m_lab + 1.0)
    pos = conf_f > 0.5
    posf = pos.astype(jnp.float32)

    g_cx = (m[0:1, :] - pcx) * inv01w
    g_cy = (m[1:2, :] - pcy) * inv01h
    g_w = m[2:3, :] - lpw5
    g_h = m[3:4, :] - lph5
    lp = loc_ref[u]
    sl1 = jnp.zeros((1, _P_PAD), jnp.float32)
    for i, g in enumerate((g_cx, g_cy, g_w, g_h)):
        d = jnp.abs(lp[i:i + 1, :] - g)
        sl1 = sl1 + jnp.where(d < 1.0, 0.5 * d * d, d - 0.5)

    x = conf_ref[u]
    xm = jnp.max(x, axis=0, keepdims=True)
    e = jnp.exp(x - xm)
    ones_r = jnp.ones((1, _NUM_CLASSES), jnp.float32)
    dn_std = (((1,), (0,)), ((), ()))
    s = lax.dot_general(ones_r, e, dn_std,
                        preferred_element_type=jnp.float32)
    lse = xm + jnp.log(s)
    iota_cf = lax.broadcasted_iota(
        jnp.int32, (_NUM_CLASSES, 1), 0).astype(jnp.float32)
    xsel_m = jnp.where(conf_f == iota_cf, x, 0.0)
    x_sel = lax.dot_general(ones_r, xsel_m, dn_std,
                            preferred_element_type=jnp.float32)
    ce_pos = (lse - x_sel) * posf

    valid = iota_p < _P
    loss_gt = jnp.where(jnp.logical_and(valid, jnp.logical_not(pos)),
                        lse - x[0:1, :], 0.0)

    lg_ref[pl.ds(bb, 1), :] = loss_gt

    stack3 = jnp.concatenate([sl1 * posf, ce_pos, posf], axis=0)
    sums3 = jnp.sum(stack3, axis=1, keepdims=True)
    np_ref[pl.ds(bb, 1), :] = jnp.broadcast_to(sums3[2:3, 0:1],
                                               (1, _LANES))
    blk = jnp.concatenate(
        [sums3, jnp.zeros((5, 1), jnp.float32)], axis=0)
    c = lax.broadcasted_iota(jnp.int32, (8, _LANES), 1)
    return jnp.where(c == 0, blk, 0.0)


def _mbl_kernel(tb_ref, tc_ref, pri_ref, loc_ref, conf_ref, out_ref,
                lg_ref, np_ref):
    b = pl.program_id(0)
    nb = pl.num_programs(0)

    total_contrib = jnp.zeros((8, _LANES), jnp.float32)
    for u in range(_BPS):
        total_contrib = total_contrib + _one_batch(
            _BPS * b + u, u, tb_ref, tc_ref, pri_ref, loc_ref, conf_ref,
            lg_ref, np_ref)

    @pl.when(b == 0)
    def _():
        out_ref[...] = total_contrib

    @pl.when(b > 0)
    def _():
        out_ref[...] = out_ref[...] + total_contrib

    @pl.when(b == nb - 1)
    def _():
        r = lax.broadcasted_iota(jnp.int32, (8, _LANES), 0)
        c = lax.broadcasted_iota(jnp.int32, (8, _LANES), 1)
        lg = lg_ref[...]
        npos = np_ref[...][:, 0:1].astype(jnp.int32)
        kk = jnp.minimum(3 * npos, _P - 1)

        def bs_body(_, carry):
            lo, hi = carry
            mid = lo + (hi - lo) // 2
            mid_f = lax.bitcast_convert_type(mid, jnp.float32)
            cnt = jnp.sum((lg > mid_f).astype(jnp.int32),
                          axis=1, keepdims=True)
            below = cnt < kk
            return (jnp.where(below, lo, mid + 1),
                    jnp.where(below, mid, hi))

        lo0 = jnp.zeros((_B, 1), jnp.int32)
        hi0 = jnp.full((_B, 1), 0x7F800000, jnp.int32)
        _, t_bits = lax.fori_loop(0, 31, bs_body, (lo0, hi0))
        t = lax.bitcast_convert_type(t_bits, jnp.float32)
        gt = lg > t
        cnt_gt = jnp.sum(gt.astype(jnp.int32), axis=1, keepdims=True)
        rows = (jnp.sum(jnp.where(gt, lg, 0.0), axis=1, keepdims=True)
                + (kk - cnt_gt).astype(jnp.float32) * t)
        rows = jnp.where(kk > 0, rows, 0.0)
        topk_total = jnp.sum(rows)
        out_ref[...] = out_ref[...] + jnp.where((r == 1) & (c == 0),
                                                topk_total, 0.0)


@functools.partial(jax.jit, static_argnames=("interpret",))
def kernel(loc_preds, conf_preds, priors, target_boxes, target_classes,
           interpret=False):
    B, P, C = conf_preds.shape
    pad = _P_PAD - P

    tb = jnp.zeros((B, _T, _LANES), jnp.float32).at[:, :, :4].set(target_boxes)
    tc = jnp.zeros((B, _T, _LANES), jnp.float32).at[:, :, 0:1].set(
        target_classes.astype(jnp.float32)[..., None])
    pri_t = jnp.transpose(priors, (1, 0))
    pri_pad = jnp.tile(jnp.array([[-10.0], [-10.0], [1.0], [1.0]],
                                 jnp.float32), (1, pad))
    pri_t = jnp.concatenate([pri_t, pri_pad], axis=1)
    pcx, pcy, pw, ph = (pri_t[i] for i in range(4))
    cx1, cy1 = pcx - pw / 2.0, pcy - ph / 2.0
    cx2, cy2 = pcx + pw / 2.0, pcy + ph / 2.0
    pp = jnp.stack([
        cx1, cy1, cx2, cy2,
        (cx2 - cx1) * (cy2 - cy1), pcx, pcy, 10.0 / pw, 10.0 / ph,
        5.0 * jnp.log(pw), 5.0 * jnp.log(ph),
    ])
    pp = jnp.concatenate(
        [pp, jnp.zeros((5, _P_PAD), jnp.float32)], axis=0)
    loc_t = jnp.pad(jnp.transpose(loc_preds, (0, 2, 1)),
                    ((0, 0), (0, 0), (0, pad)))
    conf_t = jnp.pad(jnp.transpose(conf_preds, (0, 2, 1)),
                     ((0, 0), (0, 0), (0, pad)))

    out = pl.pallas_call(
        _mbl_kernel,
        grid=(B // _BPS,),
        in_specs=[
            pl.BlockSpec((_BPS, _T, _LANES), lambda b: (b, 0, 0)),
            pl.BlockSpec((_BPS, _T, _LANES), lambda b: (b, 0, 0)),
            pl.BlockSpec((16, _P_PAD), lambda b: (0, 0)),
            pl.BlockSpec((_BPS, 4, _P_PAD), lambda b: (b, 0, 0)),
            pl.BlockSpec((_BPS, C, _P_PAD), lambda b: (b, 0, 0)),
        ],
        out_specs=pl.BlockSpec((8, _LANES), lambda b: (0, 0)),
        out_shape=jax.ShapeDtypeStruct((8, _LANES), jnp.float32),
        scratch_shapes=[
            pltpu.VMEM((_B, _P_PAD), jnp.float32),
            pltpu.VMEM((_B, _LANES), jnp.float32),
        ],
        interpret=interpret,
    )(tb, tc, pp, loc_t, conf_t)

    n = out[2, 0]
    return out[0, 0] / n, out[1, 0] / n


# speedup vs baseline: 1.0055x; 1.0055x over previous
import functools

import jax
import jax.numpy as jnp
from jax import lax
from jax.experimental import pallas as pl
from jax.experimental.pallas import tpu as pltpu

_NUM_CLASSES = 21
_THRESH = 0.45
_P = 8732
_T = 8
_B = 16
_LANES = 128
_P_PAD = ((_P + _LANES - 1) // _LANES) * _LANES
_BIG = 2**30
_BPS = 4

_DN = (((0,), (0,)), ((), ()))


def _one_batch(bb, u, tb_ref, tc_ref, pri_ref, loc_ref, conf_ref,
               lg_ref, np_ref):
    tbb = tb_ref[u]
    tx1 = tbb[:, 0:1]
    ty1 = tbb[:, 1:2]
    tx2 = tbb[:, 2:3]
    ty2 = tbb[:, 3:4]
    labf = tc_ref[u][:, 0:1]

    tw = tx2 - tx1
    th = ty2 - ty1
    area_t = tw * th
    qt = jnp.concatenate(
        [(tx1 + tx2) * 0.5, (ty1 + ty2) * 0.5,
         5.0 * jnp.log(tw), 5.0 * jnp.log(th), labf], axis=1)

    pp = pri_ref[...]
    px1 = pp[0:1, :]
    py1 = pp[1:2, :]
    px2 = pp[2:3, :]
    py2 = pp[3:4, :]
    area_p = pp[4:5, :]
    pcx = pp[5:6, :]
    pcy = pp[6:7, :]
    inv01w = pp[7:8, :]
    inv01h = pp[8:9, :]
    lpw5 = pp[9:10, :]
    lph5 = pp[10:11, :]

    iw = jnp.maximum(jnp.minimum(tx2, px2) - jnp.maximum(tx1, px1), 0.0)
    ih = jnp.maximum(jnp.minimum(ty2, py2) - jnp.maximum(ty1, py1), 0.0)
    inter = iw * ih
    ov = inter / (area_t + area_p - inter)

    iota_p = lax.broadcasted_iota(jnp.int32, (1, _P_PAD), 1)
    iota_t = lax.broadcasted_iota(jnp.int32, (_T, 1), 0)

    bp_max = jnp.max(ov, axis=1, keepdims=True)
    bp_idx = jnp.min(jnp.where(ov == bp_max, iota_p, _BIG),
                     axis=1, keepdims=True)
    bt_ov = jnp.max(ov, axis=0, keepdims=True)
    bt_idx = jnp.min(jnp.where(ov == bt_ov, iota_t, _BIG),
                     axis=0, keepdims=True)

    eq = bp_idx == iota_p
    forced = jnp.max(jnp.where(eq, iota_t, -1), axis=0, keepdims=True)
    bt_ov = jnp.where(forced >= 0, 2.0, bt_ov)
    bt_idx = jnp.where(forced >= 0, forced, bt_idx)

    onehot = jnp.where(bt_idx == iota_t, 1.0, 0.0)
    m = lax.dot_general(qt, onehot, _DN,
                        preferred_element_type=jnp.float32)
    m_lab = jnp.floor(m[4:5, :] + 0.5)
    conf_f = jnp.where(bt_ov < _THRESH, 0.0, m_lab + 1.0)
    pos = conf_f > 0.5
    posf = pos.astype(jnp.float32)

    g_cx = (m[0:1, :] - pcx) * inv01w
    g_cy = (m[1:2, :] - pcy) * inv01h
    g_w = m[2:3, :] - lpw5
    g_h = m[3:4, :] - lph5
    lp = loc_ref[u]
    sl1 = jnp.zeros((1, _P_PAD), jnp.float32)
    for i, g in enumerate((g_cx, g_cy, g_w, g_h)):
        d = jnp.abs(lp[i:i + 1, :] - g)
        sl1 = sl1 + jnp.where(d < 1.0, 0.5 * d * d, d - 0.5)

    x = conf_ref[u]
    xm = jnp.max(x, axis=0, keepdims=True)
    e = jnp.exp(x - xm)
    ones_r = jnp.ones((1, _NUM_CLASSES), jnp.float32)
    dn_std = (((1,), (0,)), ((), ()))
    s = lax.dot_general(ones_r, e, dn_std,
                        preferred_element_type=jnp.float32)
    lse = xm + jnp.log(s)
    iota_cf = lax.broadcasted_iota(
        jnp.int32, (_NUM_CLASSES, 1), 0).astype(jnp.float32)
    xsel_m = jnp.where(conf_f == iota_cf, x, 0.0)
    x_sel = lax.dot_general(ones_r, xsel_m, dn_std,
                            preferred_element_type=jnp.float32)
    ce_pos = (lse - x_sel) * posf

    valid = iota_p < _P
    loss_gt = jnp.where(jnp.logical_and(valid, jnp.logical_not(pos)),
                        lse - x[0:1, :], 0.0)

    lg_ref[pl.ds(bb, 1), :] = loss_gt

    stack3 = jnp.concatenate([sl1 * posf, ce_pos, posf], axis=0)
    sums3 = jnp.sum(stack3, axis=1, keepdims=True)
    np_ref[pl.ds(bb, 1), :] = jnp.broadcast_to(sums3[2:3, 0:1],
                                               (1, _LANES))
    blk = jnp.concatenate(
        [sums3, jnp.zeros((5, 1), jnp.float32)], axis=0)
    c = lax.broadcasted_iota(jnp.int32, (8, _LANES), 1)
    return jnp.where(c == 0, blk, 0.0)


def _mbl_kernel(tb_ref, tc_ref, pri_ref, loc_ref, conf_ref, out_ref,
                lg_ref, np_ref):
    b = pl.program_id(0)
    nb = pl.num_programs(0)

    total_contrib = jnp.zeros((8, _LANES), jnp.float32)
    for u in range(_BPS):
        total_contrib = total_contrib + _one_batch(
            _BPS * b + u, u, tb_ref, tc_ref, pri_ref, loc_ref, conf_ref,
            lg_ref, np_ref)

    @pl.when(b == 0)
    def _():
        out_ref[...] = total_contrib

    @pl.when(b > 0)
    def _():
        out_ref[...] = out_ref[...] + total_contrib

    @pl.when(b == nb - 1)
    def _():
        r = lax.broadcasted_iota(jnp.int32, (8, _LANES), 0)
        c = lax.broadcasted_iota(jnp.int32, (8, _LANES), 1)
        lg = lg_ref[...]
        npos = np_ref[...][:, 0:1].astype(jnp.int32)
        kk = jnp.minimum(3 * npos, _P - 1)

        def bs_body(_, carry):
            lo, hi = carry
            mid = lo + (hi - lo) // 2
            mid_f = lax.bitcast_convert_type(mid, jnp.float32)
            cnt = jnp.sum((lg > mid_f).astype(jnp.int32),
                          axis=1, keepdims=True)
            below = cnt < kk
            return (jnp.where(below, lo, mid + 1),
                    jnp.where(below, mid, hi))

        lo0 = jnp.zeros((_B, 1), jnp.int32)
        hi0 = jnp.full((_B, 1), 0x7F800000, jnp.int32)
        _, t_bits = lax.fori_loop(0, 31, bs_body, (lo0, hi0))
        t = lax.bitcast_convert_type(t_bits, jnp.float32)
        gt = lg > t
        cnt_gt = jnp.sum(gt.astype(jnp.int32), axis=1, keepdims=True)
        rows = (jnp.sum(jnp.where(gt, lg, 0.0), axis=1, keepdims=True)
                + (kk - cnt_gt).astype(jnp.float32) * t)
        rows = jnp.where(kk > 0, rows, 0.0)
        topk_total = jnp.sum(rows)
        out_ref[...] = out_ref[...] + jnp.where((r == 1) & (c == 0),
                                                topk_total, 0.0)


@functools.partial(jax.jit, static_argnames=("interpret",))
def kernel(loc_preds, conf_preds, priors, target_boxes, target_classes,
           interpret=False):
    B, P, C = conf_preds.shape
    pad = _P_PAD - P

    tb = jnp.zeros((B, _T, _LANES), jnp.float32).at[:, :, :4].set(target_boxes)
    tc = jnp.zeros((B, _T, _LANES), jnp.float32).at[:, :, 0:1].set(
        target_classes.astype(jnp.float32)[..., None])
    pri_t = jnp.transpose(priors, (1, 0))
    pri_pad = jnp.tile(jnp.array([[-10.0], [-10.0], [1.0], [1.0]],
                                 jnp.float32), (1, pad))
    pri_t = jnp.concatenate([pri_t, pri_pad], axis=1)
    pcx, pcy, pw, ph = (pri_t[i] for i in range(4))
    cx1, cy1 = pcx - pw / 2.0, pcy - ph / 2.0
    cx2, cy2 = pcx + pw / 2.0, pcy + ph / 2.0
    pp = jnp.stack([
        cx1, cy1, cx2, cy2,
        (cx2 - cx1) * (cy2 - cy1), pcx, pcy, 10.0 / pw, 10.0 / ph,
        5.0 * jnp.log(pw), 5.0 * jnp.log(ph),
    ])
    pp = jnp.concatenate(
        [pp, jnp.zeros((5, _P_PAD), jnp.float32)], axis=0)
    loc_t = jnp.pad(jnp.transpose(loc_preds, (0, 2, 1)),
                    ((0, 0), (0, 0), (0, pad)))
    conf_t = jnp.pad(jnp.transpose(conf_preds, (0, 2, 1)),
                     ((0, 0), (0, 0), (0, pad)))

    out = pl.pallas_call(
        _mbl_kernel,
        grid=(B // _BPS,),
        in_specs=[
            pl.BlockSpec((_BPS, _T, _LANES), lambda b: (b, 0, 0)),
            pl.BlockSpec((_BPS, _T, _LANES), lambda b: (b, 0, 0)),
            pl.BlockSpec((16, _P_PAD), lambda b: (0, 0)),
            pl.BlockSpec((_BPS, 4, _P_PAD), lambda b: (b, 0, 0)),
            pl.BlockSpec((_BPS, C, _P_PAD), lambda b: (b, 0, 0)),
        ],
        out_specs=pl.BlockSpec((8, _LANES), lambda b: (0, 0)),
        out_shape=jax.ShapeDtypeStruct((8, _LANES), jnp.float32),
        scratch_shapes=[
            pltpu.VMEM((_B, _P_PAD), jnp.float32),
            pltpu.VMEM((_B, _LANES), jnp.float32),
        ],
        interpret=interpret,
    )(tb, tc, pp, loc_t, conf_t)

    n = out[2, 0]
    return out[0, 0] / n, out[1, 0] / n


# speedup vs baseline: 1.0066x; 1.0012x over previous
import jax
import jax.numpy as jnp
from jax import lax
from jax.experimental import pallas as pl
from jax.experimental.pallas import tpu as pltpu

_NUM_CLASSES = 21
_THRESH = 0.45
_P = 8732
_T = 8
_B = 16
_LANES = 128
_P_PAD = ((_P + _LANES - 1) // _LANES) * _LANES
_BIG = 2**30
_BPS = 4

_DN = (((0,), (0,)), ((), ()))


def _one_batch(bb, u, tb_ref, tc_ref, pri_ref, loc_ref, conf_ref,
               lg_ref, np_ref):
    tbb = tb_ref[u]
    tx1 = tbb[:, 0:1]
    ty1 = tbb[:, 1:2]
    tx2 = tbb[:, 2:3]
    ty2 = tbb[:, 3:4]
    labf = tc_ref[u][:, 0:1]

    tw = tx2 - tx1
    th = ty2 - ty1
    area_t = tw * th
    qt = jnp.concatenate(
        [(tx1 + tx2) * 0.5, (ty1 + ty2) * 0.5,
         5.0 * jnp.log(tw), 5.0 * jnp.log(th), labf], axis=1)

    pp = pri_ref[...]
    px1 = pp[0:1, :]
    py1 = pp[1:2, :]
    px2 = pp[2:3, :]
    py2 = pp[3:4, :]
    area_p = pp[4:5, :]
    pcx = pp[5:6, :]
    pcy = pp[6:7, :]
    inv01w = pp[7:8, :]
    inv01h = pp[8:9, :]
    lpw5 = pp[9:10, :]
    lph5 = pp[10:11, :]

    iw = jnp.maximum(jnp.minimum(tx2, px2) - jnp.maximum(tx1, px1), 0.0)
    ih = jnp.maximum(jnp.minimum(ty2, py2) - jnp.maximum(ty1, py1), 0.0)
    inter = iw * ih
    ov = inter / (area_t + area_p - inter)

    iota_p = lax.broadcasted_iota(jnp.int32, (1, _P_PAD), 1)
    iota_t = lax.broadcasted_iota(jnp.int32, (_T, 1), 0)

    bp_max = jnp.max(ov, axis=1, keepdims=True)
    bp_idx = jnp.min(jnp.where(ov == bp_max, iota_p, _BIG),
                     axis=1, keepdims=True)
    bt_ov = jnp.max(ov, axis=0, keepdims=True)
    bt_idx = jnp.min(jnp.where(ov == bt_ov, iota_t, _BIG),
                     axis=0, keepdims=True)

    eq = bp_idx == iota_p
    forced = jnp.max(jnp.where(eq, iota_t, -1), axis=0, keepdims=True)
    bt_ov = jnp.where(forced >= 0, 2.0, bt_ov)
    bt_idx = jnp.where(forced >= 0, forced, bt_idx)

    onehot = jnp.where(bt_idx == iota_t, 1.0, 0.0)
    m = lax.dot_general(qt, onehot, _DN,
                        preferred_element_type=jnp.float32)
    m_lab = jnp.floor(m[4:5, :] + 0.5)
    conf_f = jnp.where(bt_ov < _THRESH, 0.0, m_lab + 1.0)
    pos = conf_f > 0.5
    posf = pos.astype(jnp.float32)

    g_cx = (m[0:1, :] - pcx) * inv01w
    g_cy = (m[1:2, :] - pcy) * inv01h
    g_w = m[2:3, :] - lpw5
    g_h = m[3:4, :] - lph5
    lp = loc_ref[u]
    sl1 = jnp.zeros((1, _P_PAD), jnp.float32)
    for i, g in enumerate((g_cx, g_cy, g_w, g_h)):
        d = jnp.abs(lp[i:i + 1, :] - g)
        sl1 = sl1 + jnp.where(d < 1.0, 0.5 * d * d, d - 0.5)

    x = conf_ref[u]
    xm = jnp.max(x, axis=0, keepdims=True)
    e = jnp.exp(x - xm)
    ones_r = jnp.ones((1, _NUM_CLASSES), jnp.float32)
    dn_std = (((1,), (0,)), ((), ()))
    s = lax.dot_general(ones_r, e, dn_std,
                        preferred_element_type=jnp.float32)
    lse = xm + jnp.log(s)
    iota_cf = lax.broadcasted_iota(
        jnp.int32, (_NUM_CLASSES, 1), 0).astype(jnp.float32)
    xsel_m = jnp.where(conf_f == iota_cf, x, 0.0)
    x_sel = lax.dot_general(ones_r, xsel_m, dn_std,
                            preferred_element_type=jnp.float32)
    ce_pos = (lse - x_sel) * posf

    valid = iota_p < _P
    loss_gt = jnp.where(jnp.logical_and(valid, jnp.logical_not(pos)),
                        lse - x[0:1, :], 0.0)

    lg_ref[pl.ds(bb, 1), :] = loss_gt

    stack3 = jnp.concatenate([sl1 * posf, ce_pos, posf], axis=0)
    sums3 = jnp.sum(stack3, axis=1, keepdims=True)
    np_ref[pl.ds(bb, 1), :] = jnp.broadcast_to(sums3[2:3, 0:1],
                                               (1, _LANES))
    blk = jnp.concatenate(
        [sums3, jnp.zeros((5, 1), jnp.float32)], axis=0)
    c = lax.broadcasted_iota(jnp.int32, (8, _LANES), 1)
    return jnp.where(c == 0, blk, 0.0)


def _mbl_kernel(tb_ref, tc_ref, pri_ref, loc_ref, conf_ref, out_ref,
                lg_ref, np_ref):
    b = pl.program_id(0)
    nb = pl.num_programs(0)

    total_contrib = jnp.zeros((8, _LANES), jnp.float32)
    for u in range(_BPS):
        total_contrib = total_contrib + _one_batch(
            _BPS * b + u, u, tb_ref, tc_ref, pri_ref, loc_ref, conf_ref,
            lg_ref, np_ref)

    @pl.when(b == 0)
    def _():
        out_ref[...] = total_contrib

    @pl.when(b > 0)
    def _():
        out_ref[...] = out_ref[...] + total_contrib

    @pl.when(b == nb - 1)
    def _():
        r = lax.broadcasted_iota(jnp.int32, (8, _LANES), 0)
        c = lax.broadcasted_iota(jnp.int32, (8, _LANES), 1)
        lg = lg_ref[...]
        npos = np_ref[...][:, 0:1].astype(jnp.int32)
        kk = jnp.minimum(3 * npos, _P - 1)

        def bs_body(_, carry):
            lo, hi = carry
            mid = lo + (hi - lo) // 2
            mid_f = lax.bitcast_convert_type(mid, jnp.float32)
            cnt = jnp.sum((lg > mid_f).astype(jnp.int32),
                          axis=1, keepdims=True)
            below = cnt < kk
            return (jnp.where(below, lo, mid + 1),
                    jnp.where(below, mid, hi))

        lo0 = jnp.zeros((_B, 1), jnp.int32)
        hi0 = jnp.full((_B, 1), 0x7F800000, jnp.int32)
        _, t_bits = lax.fori_loop(0, 31, bs_body, (lo0, hi0))
        t = lax.bitcast_convert_type(t_bits, jnp.float32)
        gt = lg > t
        cnt_gt = jnp.sum(gt.astype(jnp.int32), axis=1, keepdims=True)
        rows = (jnp.sum(jnp.where(gt, lg, 0.0), axis=1, keepdims=True)
                + (kk - cnt_gt).astype(jnp.float32) * t)
        rows = jnp.where(kk > 0, rows, 0.0)
        topk_total = jnp.sum(rows)
        out_ref[...] = out_ref[...] + jnp.where((r == 1) & (c == 0),
                                                topk_total, 0.0)


@jax.jit
def kernel(loc_preds, conf_preds, priors, target_boxes, target_classes):
    B, P, C = conf_preds.shape
    pad = _P_PAD - P

    tb = jnp.zeros((B, _T, _LANES), jnp.float32).at[:, :, :4].set(target_boxes)
    tc = jnp.zeros((B, _T, _LANES), jnp.float32).at[:, :, 0:1].set(
        target_classes.astype(jnp.float32)[..., None])
    pri_t = jnp.transpose(priors, (1, 0))
    pri_pad = jnp.tile(jnp.array([[-10.0], [-10.0], [1.0], [1.0]],
                                 jnp.float32), (1, pad))
    pri_t = jnp.concatenate([pri_t, pri_pad], axis=1)
    pcx, pcy, pw, ph = (pri_t[i] for i in range(4))
    cx1, cy1 = pcx - pw / 2.0, pcy - ph / 2.0
    cx2, cy2 = pcx + pw / 2.0, pcy + ph / 2.0
    pp = jnp.stack([
        cx1, cy1, cx2, cy2,
        (cx2 - cx1) * (cy2 - cy1), pcx, pcy, 10.0 / pw, 10.0 / ph,
        5.0 * jnp.log(pw), 5.0 * jnp.log(ph),
    ])
    pp = jnp.concatenate(
        [pp, jnp.zeros((5, _P_PAD), jnp.float32)], axis=0)
    loc_t = jnp.pad(jnp.transpose(loc_preds, (0, 2, 1)),
                    ((0, 0), (0, 0), (0, pad)))
    conf_t = jnp.pad(jnp.transpose(conf_preds, (0, 2, 1)),
                     ((0, 0), (0, 0), (0, pad)))

    out = pl.pallas_call(
        _mbl_kernel,
        grid=(B // _BPS,),
        in_specs=[
            pl.BlockSpec((_BPS, _T, _LANES), lambda b: (b, 0, 0)),
            pl.BlockSpec((_BPS, _T, _LANES), lambda b: (b, 0, 0)),
            pl.BlockSpec((16, _P_PAD), lambda b: (0, 0)),
            pl.BlockSpec((_BPS, 4, _P_PAD), lambda b: (b, 0, 0)),
            pl.BlockSpec((_BPS, C, _P_PAD), lambda b: (b, 0, 0)),
        ],
        out_specs=pl.BlockSpec((8, _LANES), lambda b: (0, 0)),
        out_shape=jax.ShapeDtypeStruct((8, _LANES), jnp.float32),
        scratch_shapes=[
            pltpu.VMEM((_B, _P_PAD), jnp.float32),
            pltpu.VMEM((_B, _LANES), jnp.float32),
        ],
    )(tb, tc, pp, loc_t, conf_t)

    n = out[2, 0]
    return out[0, 0] / n, out[1, 0] / n
